# baseline (speedup 1.0000x reference)
import functools

import jax
import jax.numpy as jnp
from jax import lax
from jax.experimental import pallas as pl
from jax.experimental.pallas import tpu as pltpu
from jax.experimental.pallas import tpu_sc as plsc

NC, NSUB, LANES = 2, 16, 16
NW = NC * NSUB


def _round_up(v, m):
    return -(-v // m) * m


@functools.lru_cache(maxsize=None)
def _sc_gather_call(V, D, B):
    per_worker = _round_up(-(-B // NW), 8)
    if per_worker <= 128:
        chunk, sub, nch = per_worker, 0, 1
    else:
        bytes_cap = 256 * 1024
        sub_max = max(1, min(8, bytes_cap // (128 * D * 4)))
        units = -(-B // (NW * 128))
        nch = -(-units // sub_max)
        sub = -(-units // nch)
        chunk = sub * 128
    b_pad = NW * chunk * nch

    mesh = plsc.VectorSubcoreMesh(
        core_axis_name="c", subcore_axis_name="s",
        num_cores=NC, num_subcores=NSUB)

    if sub == 0:
        @functools.partial(
            pl.kernel,
            out_type=jax.ShapeDtypeStruct((b_pad, D), jnp.float32),
            mesh=mesh,
            compiler_params=pltpu.CompilerParams(use_tc_tiling_on_sc=False),
            scratch_types=[
                pltpu.VMEM((chunk,), jnp.int32),
                pltpu.VMEM((chunk, D), jnp.float32),
                pltpu.SemaphoreType.DMA,
            ],
            name=f"sc_gather_s_{V}_{D}_{B}",
        )
        def gather_small(table_hbm, idx_hbm, out_hbm, idx_v, rows_v, sem):
            wid = lax.axis_index("s") * NC + lax.axis_index("c")
            off = pl.multiple_of(wid * chunk, 8)
            pltpu.sync_copy(idx_hbm.at[pl.ds(off, chunk)], idx_v)
            pltpu.async_copy(table_hbm.at[idx_v], rows_v, sem).wait()
            pltpu.sync_copy(rows_v, out_hbm.at[pl.ds(off, chunk), :])

        def call(table, idx_pad):
            return gather_small(table, idx_pad)

        return call, b_pad

    @functools.partial(
        pl.kernel,
        out_type=jax.ShapeDtypeStruct((b_pad, D), jnp.float32),
        mesh=mesh,
        compiler_params=pltpu.CompilerParams(use_tc_tiling_on_sc=False),
        scratch_types=[
            pltpu.VMEM((sub, 128), jnp.int32),
            pltpu.VMEM((chunk, D), jnp.float32),
            pltpu.SemaphoreType.DMA,
        ],
        name=f"sc_gather_{V}_{D}_{B}",
    )
    def gather_big(table_hbm, idx_hbm, out_hbm, idx_v, rows_v, sem):
        wid = lax.axis_index("s") * NC + lax.axis_index("c")

        def body(ci, carry):
            off = pl.multiple_of((wid * nch + ci) * chunk, 128)
            pltpu.sync_copy(idx_hbm.at[pl.ds(pl.multiple_of(off // 128, 8),
                                             sub), :], idx_v)
            descs = []
            for k in range(sub):
                descs.append(pltpu.async_copy(
                    table_hbm.at[idx_v.at[k]],
                    rows_v.at[pl.ds(k * 128, 128), :], sem))
            for d in descs:
                d.wait()
            pltpu.sync_copy(rows_v, out_hbm.at[pl.ds(off, chunk), :])
            return carry

        lax.fori_loop(0, nch, body, 0)

    def call(table, idx_pad):
        return gather_big(table, idx_pad.reshape(b_pad // 128, 128))

    return call, b_pad


def _sc_gather(table, idx):
    V, D = table.shape
    B = idx.shape[0]
    call, b_pad = _sc_gather_call(V, D, B)
    if b_pad > B:
        idx = jnp.concatenate([idx, jnp.zeros((b_pad - B,), jnp.int32)])
    out = call(table, idx)
    return out[:B]


_PREC = lax.Precision.HIGHEST
_EPS = 1e-5


def _dot(a, b):
    return jnp.dot(a, b, preferred_element_type=jnp.float32, precision=_PREC)


def _dot_ref(a, b):
    return jnp.dot(a, b, preferred_element_type=jnp.float32)


def _gn_lrelu(h, ga, be):
    cout = h.shape[1]
    cs = cout // 4
    parts = []
    for g in range(4):
        hg = h[:, g * cs:(g + 1) * cs]
        mu = jnp.mean(hg, axis=1, keepdims=True)
        var = jnp.mean(hg * hg, axis=1, keepdims=True) - mu * mu
        parts.append((hg - mu) * lax.rsqrt(var + _EPS))
    y = jnp.concatenate(parts, axis=1) * ga + be
    return jnp.where(y >= 0, y, 0.2 * y)


def _conv_body(g_ref, w_ref, b_ref, ga_ref, be_ref, o_ref):
    h = _dot_ref(g_ref[...], w_ref[...]) + b_ref[...]
    o_ref[...] = _gn_lrelu(h, ga_ref[...], be_ref[...])


def _onehot_count(idx2d, nsrc):
    row = lax.broadcasted_iota(jnp.int32, (1, nsrc), 1)
    acc = jnp.zeros((idx2d.shape[0], nsrc), jnp.float32)
    for k in range(7):
        acc = acc + jnp.where(idx2d[:, k:k + 1] == row, 1.0, 0.0)
    return acc


def _onehot_conv(x, idx2d, w, b, ga, be):
    nsrc, c = x.shape
    r = idx2d.shape[0]
    row = lax.broadcasted_iota(jnp.int32, (1, nsrc), 1)
    ohv = jnp.concatenate(
        [jnp.where(idx2d[:, k:k + 1] == row, 1.0, 0.0) for k in range(7)],
        axis=0)
    p = _dot(ohv, x)
    g = jnp.concatenate([p[k * r:(k + 1) * r, :] for k in range(7)], axis=1)
    h = _dot_ref(g, w) + b
    return _gn_lrelu(h, ga, be)


def _onehot_level(x, p_ref, c_ref, wa, ba, ga, ea, wb, bb, gb, eb):
    nsrc = x.shape[0]
    cnt = _onehot_count(p_ref[...], nsrc)
    x = (1.0 / 7.0) * _dot(cnt, x)
    x = _onehot_conv(x, c_ref[...], wa[...], ba[...], ga[...], ea[...])
    return _onehot_conv(x, c_ref[...], wb[...], bb[...], gb[...], eb[...])


def _lvl3_body(x_ref, p3_ref, c3_ref,
               w3a_ref, b3a_ref, g3a_ref, e3a_ref,
               w3b_ref, b3b_ref, g3b_ref, e3b_ref, o_ref):
    o_ref[...] = _onehot_level(x_ref[...], p3_ref, c3_ref,
                               w3a_ref, b3a_ref, g3a_ref, e3a_ref,
                               w3b_ref, b3b_ref, g3b_ref, e3b_ref)


def _tail_body(x_ref, p4_ref, c4_ref, p5_ref, c5_ref,
               w4a_ref, b4a_ref, g4a_ref, e4a_ref,
               w4b_ref, b4b_ref, g4b_ref, e4b_ref,
               w5a_ref, b5a_ref, g5a_ref, e5a_ref,
               w5b_ref, b5b_ref, g5b_ref, e5b_ref,
               wo_ref, bo_ref, lat_ref, out_ref):
    x = _onehot_level(x_ref[...], p4_ref, c4_ref,
                      w4a_ref, b4a_ref, g4a_ref, e4a_ref,
                      w4b_ref, b4b_ref, g4b_ref, e4b_ref)
    x = _onehot_level(x, p5_ref, c5_ref,
                      w5a_ref, b5a_ref, g5a_ref, e5a_ref,
                      w5b_ref, b5b_ref, g5b_ref, e5b_ref)
    m = jnp.mean(x, axis=1, keepdims=True)
    lat_ref[...] = m
    o = jnp.sum(m * wo_ref[...], axis=0, keepdims=True) + bo_ref[...]
    out_ref[...] = 1.0 / (1.0 + jnp.exp(-o))


def _pool_body(p_ref, o_ref):
    c = o_ref.shape[1]
    p = p_ref[...]
    acc = p[:, :c]
    for k in range(1, 7):
        acc = acc + p[:, k * c:(k + 1) * c]
    o_ref[...] = acc * (1.0 / 7.0)


def _row_blocks(n):
    bn = min(4096, _round_up(n, 8))
    return bn, -(-n // bn)


def _conv_tc(g2, w, b, ga, be):
    n, k = g2.shape
    cout = w.shape[1]
    bn, grid = _row_blocks(n)
    row = pl.BlockSpec((bn, k), lambda i: (i, 0))
    full = lambda s: pl.BlockSpec(s, lambda i: (0, 0))
    return pl.pallas_call(
        _conv_body,
        grid=(grid,),
        in_specs=[row, full((k, cout)), full((1, cout)), full((1, cout)),
                  full((1, cout))],
        out_specs=pl.BlockSpec((bn, cout), lambda i: (i, 0)),
        out_shape=jax.ShapeDtypeStruct((n, cout), jnp.float32),
    )(g2, w, b, ga, be)


def _pool_tc(p2):
    n, k7 = p2.shape
    c = k7 // 7
    bn, grid = _row_blocks(n)
    return pl.pallas_call(
        _pool_body,
        grid=(grid,),
        in_specs=[pl.BlockSpec((bn, k7), lambda i: (i, 0))],
        out_specs=pl.BlockSpec((bn, c), lambda i: (i, 0)),
        out_shape=jax.ShapeDtypeStruct((n, c), jnp.float32),
    )(p2)


_NS = [40962, 10242, 2562, 642, 162, 42]


def kernel(x, no0, no1, no2, no3, no4, no5,
           W0a, b0a, g0a, be0a, W0b, b0b, g0b, be0b,
           W1a, b1a, g1a, be1a, W1b, b1b, g1b, be1b,
           W2a, b2a, g2a, be2a, W2b, b2b, g2b, be2b,
           W3a, b3a, g3a, be3a, W3b, b3b, g3b, be3b,
           W4a, b4a, g4a, be4a, W4b, b4b, g4b, be4b,
           W5a, b5a, g5a, be5a, W5b, b5b, g5b, be5b,
           Wout, bout):
    nos = [no0, no1, no2, no3, no4, no5]
    prm = {}
    for i, (Wa, ba, ga, bea, Wb, bb, gb, beb) in enumerate([
            (W0a, b0a, g0a, be0a, W0b, b0b, g0b, be0b),
            (W1a, b1a, g1a, be1a, W1b, b1b, g1b, be1b),
            (W2a, b2a, g2a, be2a, W2b, b2b, g2b, be2b),
            (W3a, b3a, g3a, be3a, W3b, b3b, g3b, be3b),
            (W4a, b4a, g4a, be4a, W4b, b4b, g4b, be4b),
            (W5a, b5a, g5a, be5a, W5b, b5b, g5b, be5b)]):
        prm[i] = {"a": (Wa, ba.reshape(1, -1), ga.reshape(1, -1),
                        bea.reshape(1, -1)),
                  "b": (Wb, bb.reshape(1, -1), gb.reshape(1, -1),
                        beb.reshape(1, -1))}

    xcur = x
    for i in range(3):
        n = _NS[i]
        if i > 0:
            c = xcur.shape[1]
            pooled = _sc_gather(xcur, nos[i - 1][: n * 7])
            xcur = _pool_tc(pooled.reshape(n, 7 * c))
        for s in ("a", "b"):
            cin = xcur.shape[1]
            g = _sc_gather(xcur, nos[i]).reshape(n, 7 * cin)
            W, b, ga, be = prm[i][s]
            xcur = _conv_tc(g, W, b, ga, be)

    idx_args = []
    for i in (3, 4, 5):
        n = _NS[i]
        idx_args.append(nos[i - 1][: n * 7].reshape(n, 7))
        idx_args.append(nos[i].reshape(n, 7))
    x3 = pl.pallas_call(
        _lvl3_body,
        out_shape=jax.ShapeDtypeStruct((642, 128), jnp.float32),
    )(xcur, *idx_args[:2], *prm[3]["a"], *prm[3]["b"])
    w_args = []
    for i in (4, 5):
        for s in ("a", "b"):
            w_args.extend(prm[i][s])
    lat, out = pl.pallas_call(
        _tail_body,
        out_shape=(jax.ShapeDtypeStruct((42, 1), jnp.float32),
                   jax.ShapeDtypeStruct((1, 4), jnp.float32)),
    )(x3, *idx_args[2:], *w_args, Wout, bout.reshape(1, 4))
    return out, lat.reshape(1, 42)


# speedup vs baseline: 1.0899x; 1.0899x over previous
import functools

import jax
import jax.numpy as jnp
from jax import lax
from jax.experimental import pallas as pl
from jax.experimental.pallas import tpu as pltpu
from jax.experimental.pallas import tpu_sc as plsc

NC, NSUB, LANES = 2, 16, 16
NW = NC * NSUB


def _round_up(v, m):
    return -(-v // m) * m


@functools.lru_cache(maxsize=None)
def _sc_gather_call(V, D, B):
    per_worker = _round_up(-(-B // NW), 8)
    if per_worker <= 128:
        chunk, sub, nch = per_worker, 0, 1
    else:
        bytes_cap = 256 * 1024
        sub_max = max(1, min(8, bytes_cap // (128 * D * 4)))
        units = -(-B // (NW * 128))
        nch = -(-units // sub_max)
        sub = -(-units // nch)
        chunk = sub * 128
    b_pad = NW * chunk * nch

    mesh = plsc.VectorSubcoreMesh(
        core_axis_name="c", subcore_axis_name="s",
        num_cores=NC, num_subcores=NSUB)

    if sub == 0:
        @functools.partial(
            pl.kernel,
            out_type=jax.ShapeDtypeStruct((b_pad, D), jnp.float32),
            mesh=mesh,
            compiler_params=pltpu.CompilerParams(use_tc_tiling_on_sc=False),
            scratch_types=[
                pltpu.VMEM((chunk,), jnp.int32),
                pltpu.VMEM((chunk, D), jnp.float32),
                pltpu.SemaphoreType.DMA,
            ],
            name=f"sc_gather_s_{V}_{D}_{B}",
        )
        def gather_small(table_hbm, idx_hbm, out_hbm, idx_v, rows_v, sem):
            wid = lax.axis_index("s") * NC + lax.axis_index("c")
            off = pl.multiple_of(wid * chunk, 8)
            pltpu.sync_copy(idx_hbm.at[pl.ds(off, chunk)], idx_v)
            pltpu.async_copy(table_hbm.at[idx_v], rows_v, sem).wait()
            pltpu.sync_copy(rows_v, out_hbm.at[pl.ds(off, chunk), :])

        def call(table, idx_pad):
            return gather_small(table, idx_pad)

        return call, b_pad

    @functools.partial(
        pl.kernel,
        out_type=jax.ShapeDtypeStruct((b_pad, D), jnp.float32),
        mesh=mesh,
        compiler_params=pltpu.CompilerParams(use_tc_tiling_on_sc=False),
        scratch_types=[
            pltpu.VMEM((sub, 128), jnp.int32),
            pltpu.VMEM((chunk, D), jnp.float32),
            pltpu.SemaphoreType.DMA,
        ],
        name=f"sc_gather_{V}_{D}_{B}",
    )
    def gather_big(table_hbm, idx_hbm, out_hbm, idx_v, rows_v, sem):
        wid = lax.axis_index("s") * NC + lax.axis_index("c")

        def body(ci, carry):
            off = pl.multiple_of((wid * nch + ci) * chunk, 128)
            pltpu.sync_copy(idx_hbm.at[pl.ds(pl.multiple_of(off // 128, 8),
                                             sub), :], idx_v)
            descs = []
            for k in range(sub):
                descs.append(pltpu.async_copy(
                    table_hbm.at[idx_v.at[k]],
                    rows_v.at[pl.ds(k * 128, 128), :], sem))
            for d in descs:
                d.wait()
            pltpu.sync_copy(rows_v, out_hbm.at[pl.ds(off, chunk), :])
            return carry

        lax.fori_loop(0, nch, body, 0)

    def call(table, idx_pad):
        return gather_big(table, idx_pad.reshape(b_pad // 128, 128))

    return call, b_pad


def _sc_gather(table, idx):
    V, D = table.shape
    B = idx.shape[0]
    call, b_pad = _sc_gather_call(V, D, B)
    if b_pad > B:
        idx = jnp.concatenate([idx, jnp.zeros((b_pad - B,), jnp.int32)])
    out = call(table, idx)
    return out[:B]


_PREC = lax.Precision.HIGHEST
_EPS = 1e-5


def _dot(a, b):
    return jnp.dot(a, b, preferred_element_type=jnp.float32, precision=_PREC)


def _dot_ref(a, b):
    return jnp.dot(a, b, preferred_element_type=jnp.float32)


def _dot_hi(a, b):
    return jnp.dot(a, b, preferred_element_type=jnp.float32)


def _gn_lrelu(h, ga, be):
    cout = h.shape[1]
    cs = cout // 4
    rid = lax.broadcasted_iota(jnp.int32, (2 * cout, 8), 0)
    cid = lax.broadcasted_iota(jnp.int32, (2 * cout, 8), 1)
    sel = (rid // cout == cid // 4) & ((rid % cout) // cs == cid % 4)
    a2 = jnp.where(sel, 1.0 / cs, 0.0)
    st = _dot(jnp.concatenate([h, h * h], axis=1), a2)
    mu4 = st[:, :4]
    var4 = st[:, 4:] - mu4 * mu4
    rid2 = lax.broadcasted_iota(jnp.int32, (8, 2 * cout), 0)
    cid2 = lax.broadcasted_iota(jnp.int32, (8, 2 * cout), 1)
    sel2 = rid2 == (cid2 // cout) * 4 + (cid2 % cout) // cs
    b2 = jnp.where(sel2, 1.0, 0.0)
    mv = _dot(jnp.concatenate([mu4, var4], axis=1), b2)
    y = (h - mv[:, :cout]) * lax.rsqrt(mv[:, cout:] + _EPS) * ga + be
    return jnp.where(y >= 0, y, 0.2 * y)


def _conv_body(g_ref, w_ref, b_ref, ga_ref, be_ref, o_ref):
    h = _dot_ref(g_ref[...], w_ref[...]) + b_ref[...]
    o_ref[...] = _gn_lrelu(h, ga_ref[...], be_ref[...])


def _onehot_count(idx2d, nsrc):
    row = lax.broadcasted_iota(jnp.int32, (1, nsrc), 1)
    acc = jnp.zeros((idx2d.shape[0], nsrc), jnp.float32)
    for k in range(7):
        acc = acc + jnp.where(idx2d[:, k:k + 1] == row, 1.0, 0.0)
    return acc


def _onehot_conv(x, idx2d, w, b, ga, be):
    nsrc, c = x.shape
    r = idx2d.shape[0]
    row = lax.broadcasted_iota(jnp.int32, (1, nsrc), 1)
    ohs = [jnp.where(idx2d[:, k:k + 1] == row, 1.0, 0.0) for k in range(7)]
    if 7 * r * nsrc * 4 <= 24 * 1024 * 1024:
        p = _dot_hi(jnp.concatenate(ohs, axis=0), x)
        g = jnp.concatenate(
            [p[k * r:(k + 1) * r, :] for k in range(7)], axis=1)
    else:
        g = jnp.concatenate([_dot_hi(oh, x) for oh in ohs], axis=1)
    h = _dot_ref(g, w) + b
    return _gn_lrelu(h, ga, be)


def _onehot_level(x, p_ref, c_ref, wa, ba, ga, ea, wb, bb, gb, eb):
    nsrc = x.shape[0]
    cnt = _onehot_count(p_ref[...], nsrc)
    x = (1.0 / 7.0) * _dot(cnt, x)
    x = _onehot_conv(x, c_ref[...], wa[...], ba[...], ga[...], ea[...])
    return _onehot_conv(x, c_ref[...], wb[...], bb[...], gb[...], eb[...])


def _lvl2_body(xp_ref, c2_ref,
               w2a_ref, b2a_ref, g2a_ref, e2a_ref,
               w2b_ref, b2b_ref, g2b_ref, e2b_ref, o_ref):
    x = _onehot_conv(xp_ref[...], c2_ref[...], w2a_ref[...], b2a_ref[...],
                     g2a_ref[...], e2a_ref[...])
    o_ref[...] = _onehot_conv(x, c2_ref[...], w2b_ref[...], b2b_ref[...],
                              g2b_ref[...], e2b_ref[...])


def _lvl3_body(x_ref, p3_ref, c3_ref,
               w3a_ref, b3a_ref, g3a_ref, e3a_ref,
               w3b_ref, b3b_ref, g3b_ref, e3b_ref, o_ref):
    o_ref[...] = _onehot_level(x_ref[...], p3_ref, c3_ref,
                               w3a_ref, b3a_ref, g3a_ref, e3a_ref,
                               w3b_ref, b3b_ref, g3b_ref, e3b_ref)


def _tail_body(x_ref, p4_ref, c4_ref, p5_ref, c5_ref,
               w4a_ref, b4a_ref, g4a_ref, e4a_ref,
               w4b_ref, b4b_ref, g4b_ref, e4b_ref,
               w5a_ref, b5a_ref, g5a_ref, e5a_ref,
               w5b_ref, b5b_ref, g5b_ref, e5b_ref,
               wo_ref, bo_ref, lat_ref, out_ref):
    x = _onehot_level(x_ref[...], p4_ref, c4_ref,
                      w4a_ref, b4a_ref, g4a_ref, e4a_ref,
                      w4b_ref, b4b_ref, g4b_ref, e4b_ref)
    x = _onehot_level(x, p5_ref, c5_ref,
                      w5a_ref, b5a_ref, g5a_ref, e5a_ref,
                      w5b_ref, b5b_ref, g5b_ref, e5b_ref)
    m = jnp.mean(x, axis=1, keepdims=True)
    lat_ref[...] = m
    o = jnp.sum(m * wo_ref[...], axis=0, keepdims=True) + bo_ref[...]
    out_ref[...] = 1.0 / (1.0 + jnp.exp(-o))


def _pool_body(p_ref, o_ref):
    c = o_ref.shape[1]
    rid = lax.broadcasted_iota(jnp.int32, (7 * c, c), 0)
    cid = lax.broadcasted_iota(jnp.int32, (7 * c, c), 1)
    avg7 = jnp.where(rid % c == cid, 1.0 / 7.0, 0.0)
    o_ref[...] = jnp.dot(p_ref[...], avg7, preferred_element_type=jnp.float32,
                         precision=_PREC)


def _row_blocks(n):
    bn = min(1024, _round_up(n, 8))
    return bn, -(-n // bn)


def _conv_tc(g2, w, b, ga, be):
    n, k = g2.shape
    cout = w.shape[1]
    bn, grid = _row_blocks(n)
    row = pl.BlockSpec((bn, k), lambda i: (i, 0))
    full = lambda s: pl.BlockSpec(s, lambda i: (0, 0))
    return pl.pallas_call(
        _conv_body,
        grid=(grid,),
        in_specs=[row, full((k, cout)), full((1, cout)), full((1, cout)),
                  full((1, cout))],
        out_specs=pl.BlockSpec((bn, cout), lambda i: (i, 0)),
        out_shape=jax.ShapeDtypeStruct((n, cout), jnp.float32),
    )(g2, w, b, ga, be)


def _pool_tc(p2):
    n, k7 = p2.shape
    c = k7 // 7
    bn, grid = _row_blocks(n)
    return pl.pallas_call(
        _pool_body,
        grid=(grid,),
        in_specs=[pl.BlockSpec((bn, k7), lambda i: (i, 0))],
        out_specs=pl.BlockSpec((bn, c), lambda i: (i, 0)),
        out_shape=jax.ShapeDtypeStruct((n, c), jnp.float32),
    )(p2)


_NS = [40962, 10242, 2562, 642, 162, 42]


def kernel(x, no0, no1, no2, no3, no4, no5,
           W0a, b0a, g0a, be0a, W0b, b0b, g0b, be0b,
           W1a, b1a, g1a, be1a, W1b, b1b, g1b, be1b,
           W2a, b2a, g2a, be2a, W2b, b2b, g2b, be2b,
           W3a, b3a, g3a, be3a, W3b, b3b, g3b, be3b,
           W4a, b4a, g4a, be4a, W4b, b4b, g4b, be4b,
           W5a, b5a, g5a, be5a, W5b, b5b, g5b, be5b,
           Wout, bout):
    nos = [no0, no1, no2, no3, no4, no5]
    prm = {}
    for i, (Wa, ba, ga, bea, Wb, bb, gb, beb) in enumerate([
            (W0a, b0a, g0a, be0a, W0b, b0b, g0b, be0b),
            (W1a, b1a, g1a, be1a, W1b, b1b, g1b, be1b),
            (W2a, b2a, g2a, be2a, W2b, b2b, g2b, be2b),
            (W3a, b3a, g3a, be3a, W3b, b3b, g3b, be3b),
            (W4a, b4a, g4a, be4a, W4b, b4b, g4b, be4b),
            (W5a, b5a, g5a, be5a, W5b, b5b, g5b, be5b)]):
        prm[i] = {"a": (Wa, ba.reshape(1, -1), ga.reshape(1, -1),
                        bea.reshape(1, -1)),
                  "b": (Wb, bb.reshape(1, -1), gb.reshape(1, -1),
                        beb.reshape(1, -1))}

    xcur = x
    for i in range(2):
        n = _NS[i]
        if i > 0:
            c = xcur.shape[1]
            pooled = _sc_gather(xcur, nos[i - 1][: n * 7])
            xcur = _pool_tc(pooled.reshape(n, 7 * c))
        for s in ("a", "b"):
            cin = xcur.shape[1]
            g = _sc_gather(xcur, nos[i]).reshape(n, 7 * cin)
            W, b, ga, be = prm[i][s]
            xcur = _conv_tc(g, W, b, ga, be)

    n2 = _NS[2]
    c = xcur.shape[1]
    pooled = _sc_gather(xcur, nos[1][: n2 * 7])
    xp2 = _pool_tc(pooled.reshape(n2, 7 * c))
    xcur = pl.pallas_call(
        _lvl2_body,
        out_shape=jax.ShapeDtypeStruct((n2, 64), jnp.float32),
    )(xp2, nos[2].reshape(n2, 7), *prm[2]["a"], *prm[2]["b"])

    idx_args = []
    for i in (3, 4, 5):
        n = _NS[i]
        idx_args.append(nos[i - 1][: n * 7].reshape(n, 7))
        idx_args.append(nos[i].reshape(n, 7))
    x3 = pl.pallas_call(
        _lvl3_body,
        out_shape=jax.ShapeDtypeStruct((642, 128), jnp.float32),
    )(xcur, *idx_args[:2], *prm[3]["a"], *prm[3]["b"])
    w_args = []
    for i in (4, 5):
        for s in ("a", "b"):
            w_args.extend(prm[i][s])
    lat, out = pl.pallas_call(
        _tail_body,
        out_shape=(jax.ShapeDtypeStruct((42, 1), jnp.float32),
                   jax.ShapeDtypeStruct((1, 4), jnp.float32)),
    )(x3, *idx_args[2:], *w_args, Wout, bout.reshape(1, 4))
    return out, lat.reshape(1, 42)
